# Initial kernel scaffold; baseline (speedup 1.0000x reference)
import jax
import jax.numpy as jnp
from jax.experimental import pallas as pl


def kernel(fourmomenta, scalars, batch, ptr, is_spurion, params):
    raise NotImplementedError("write your pallas kernel here")


# speedup vs baseline: 7.1544x; 7.1544x over previous
import jax
import jax.numpy as jnp
from jax.experimental import pallas as pl
from jax.experimental.pallas import tpu as pltpu

_P = 64
_NH = 32
_NL = 3
_CW = 0.005
_JPB = 1
_NPW = 17

_HI = jax.lax.Precision.HIGHEST


def _psi(v):
    return jnp.sign(v) * jnp.log(jnp.abs(v) + 1.0)


def _bf(v):
    return v.astype(jnp.bfloat16).astype(jnp.float32)


def _mm(a, b):
    return jax.lax.dot(a, b, precision=_HI)


def _body(fm_ref, sc_ref, sp_ref, *a):
    out_ref = a[-1]
    w = a[:-1]
    n = _JPB * _P
    e = _JPB * _P * _P

    sp = sp_ref[...]
    x = fm_ref[...] * (0.05 + 0.95 * sp)
    h = _mm(_bf(sc_ref[...]), w[0][...]) + w[1][...]

    lane4 = jax.lax.broadcasted_iota(jnp.int32, (1, 4), 1)
    metric = jnp.where(lane4 == 0, 1.0, -1.0).astype(jnp.float32)

    eidx = jax.lax.broadcasted_iota(jnp.int32, (e, 1), 0)
    mask = (((eidx >> 6) & 63) != (eidx & 63)).astype(jnp.float32)

    for l in range(_NL):
        (Wa, Wb, wn, wd, e1b, e2W, e2b, mWr, mb, x1W, x1b, x2Wr,
         h1a, h1b2, h1bias, h2W, h2b) = (
            w[2 + l * _NPW + k][...] for k in range(_NPW))

        xs = x.reshape(_JPB, _P, 1, 4)
        xi = jnp.broadcast_to(xs, (_JPB, _P, _P, 4)).reshape(e, 4)
        xj = jnp.broadcast_to(x.reshape(_JPB, 1, _P, 4),
                              (_JPB, _P, _P, 4)).reshape(e, 4)
        xd = xi - xj
        dots = _psi(jnp.sum(xi * xj * metric, axis=-1, keepdims=True))
        norms = _psi(jnp.sum(xd * xd * metric, axis=-1, keepdims=True))

        hb = _bf(h)
        A = _mm(hb, Wa)
        Bv = _mm(hb, Wb)
        pre = (jnp.broadcast_to(A.reshape(_JPB, _P, 1, _NH),
                                (_JPB, _P, _P, _NH)).reshape(e, _NH)
               + jnp.broadcast_to(Bv.reshape(_JPB, 1, _P, _NH),
                                  (_JPB, _P, _P, _NH)).reshape(e, _NH)
               + _bf(norms) * wn + _bf(dots) * wd + e1b)
        m = jnp.maximum(
            _mm(_bf(jnp.maximum(pre, 0.0)), e2W) + e2b, 0.0)
        mb16 = _bf(m)

        wgt = jax.nn.sigmoid(
            jnp.sum(mb16 * mWr, axis=-1, keepdims=True) + mb)
        u = jnp.maximum(_mm(mb16, x1W) + x1b, 0.0)
        xw = jnp.sum(_bf(u) * x2Wr, axis=-1, keepdims=True)

        trans = jnp.clip(xd * xw, -100.0, 100.0)
        tsum = jnp.sum(trans.reshape(n, _P, 4), axis=1)
        x = x + _CW * (tsum / 63.0)

        mw = m * wgt * mask
        hagg = jnp.sum(mw.reshape(n, _P, _NH), axis=1)
        hu = jnp.maximum(_mm(hb, h1a) + _mm(_bf(hagg), h1b2) + h1bias, 0.0)
        h = h + _mm(_bf(hu), h2W) + h2b

    hm = jnp.sum(h.reshape(_JPB, _P, _NH), axis=1) * (1.0 / _P)
    o = (_mm(_bf(jnp.maximum(_mm(_bf(hm), w[-4][...]) + w[-3][...], 0.0)),
             w[-2][...]) + w[-1][...])
    out_ref[...] = o.reshape(out_ref.shape)


def _full(shape):
    nd = len(shape)
    return pl.BlockSpec(shape, lambda i, _nd=nd: (0,) * _nd)


def kernel(fourmomenta, scalars, batch, ptr, is_spurion, params):
    N = fourmomenta.shape[0]
    B = ptr.shape[0] - 1
    p = params

    def r(v):
        return v.astype(jnp.bfloat16).astype(jnp.float32)

    wl = [r(p['embed_W']), p['embed_b'].reshape(1, -1)]
    for l in range(_NL):
        e1W = r(p[f'e1_W_{l}'])
        h1W = r(p[f'h1_W_{l}'])
        wl += [
            e1W[0:_NH], e1W[_NH:2 * _NH], e1W[2 * _NH:2 * _NH + 1],
            e1W[2 * _NH + 1:2 * _NH + 2], p[f'e1_b_{l}'].reshape(1, -1),
            r(p[f'e2_W_{l}']), p[f'e2_b_{l}'].reshape(1, -1),
            r(p[f'm_W_{l}']).reshape(1, -1), p[f'm_b_{l}'].reshape(1, -1),
            r(p[f'x1_W_{l}']), p[f'x1_b_{l}'].reshape(1, -1),
            r(p[f'x2_W_{l}']).reshape(1, -1),
            h1W[0:_NH], h1W[_NH:2 * _NH], p[f'h1_b_{l}'].reshape(1, -1),
            r(p[f'h2_W_{l}']), p[f'h2_b_{l}'].reshape(1, -1),
        ]
    wl += [r(p['dec1_W']), p['dec1_b'].reshape(1, -1),
           r(p['dec2_W']), p['dec2_b'].reshape(1, -1)]

    sp = is_spurion.astype(jnp.float32).reshape(N, 1)
    npb = _JPB * _P

    in_specs = [
        pl.BlockSpec((npb, 4), lambda i: (i, 0)),
        pl.BlockSpec((npb, scalars.shape[1]), lambda i: (i, 0)),
        pl.BlockSpec((npb, 1), lambda i: (i, 0)),
    ] + [_full(wi.shape) for wi in wl]

    out = pl.pallas_call(
        _body,
        grid=(B // _JPB,),
        in_specs=in_specs,
        out_specs=pl.BlockSpec((1, _JPB, 2), lambda i: (i, 0, 0)),
        out_shape=jax.ShapeDtypeStruct((B // _JPB, _JPB, 2), jnp.float32),
        compiler_params=pltpu.CompilerParams(
            dimension_semantics=("parallel",)),
    )(fourmomenta, scalars, sp, *wl)
    return out.reshape(B, 2)
